# baseline (device time: 374711 ns/iter reference)
import jax
import jax.numpy as jnp
from jax import lax
from jax.experimental import pallas as pl
from jax.experimental.pallas import tpu as pltpu

N_DEV = 32
N_EXP_LOCAL = 4
N_SUB = 2
N_STEP = (N_DEV - 1) * N_SUB


def kernel(x, router_W, route_idx, expert_W):
    n_tok, d_model = x.shape
    d_ff = expert_W.shape[-1]
    n_exp = router_W.shape[-1]

    ew_bf = expert_W.astype(jnp.bfloat16)

    def body(x_ref, rw_ref, idx_ref, ew_ref, out_ref, w_lh,
             w_r, w_lv, gate_s,
             send_r, recv_r, send_l, recv_l, loc_sems):
        me = lax.axis_index("i")
        left = lax.rem(me + N_DEV - 1, N_DEV)
        right = lax.rem(me + 1, N_DEV)

        barrier_sem = pltpu.get_barrier_semaphore()
        for nbr in (left, right):
            pl.semaphore_signal(
                barrier_sem, inc=1,
                device_id=(nbr,), device_id_type=pl.DeviceIdType.MESH,
            )
        pl.semaphore_wait(barrier_sem, 2)

        for k in range(N_SUB):
            w_r[N_SUB * me + k] = ew_ref[k]
            w_lv[N_SUB * me + k] = ew_ref[N_SUB + k]

        def fwd_r(g, k, do_start):
            o = lax.rem(me - g + N_DEV, N_DEV)
            slot = N_SUB * o + k
            r = pltpu.make_async_remote_copy(
                src_ref=w_r.at[slot], dst_ref=w_r.at[slot],
                send_sem=send_r.at[N_SUB * g + k],
                recv_sem=recv_r.at[N_SUB * g + k],
                device_id=(right,), device_id_type=pl.DeviceIdType.MESH,
            )
            if do_start:
                r.start()
            return r

        def fwd_l(g, k, do_start):
            o = lax.rem(me + g, N_DEV)
            slot = N_SUB * o + k
            src = w_lv.at[slot] if g == 0 else w_lh.at[slot]
            r = pltpu.make_async_remote_copy(
                src_ref=src, dst_ref=w_lh.at[slot],
                send_sem=send_l.at[N_SUB * g + k],
                recv_sem=recv_l.at[N_SUB * g + k],
                device_id=(left,), device_id_type=pl.DeviceIdType.MESH,
            )
            if do_start:
                r.start()
            return r

        for k in range(N_SUB):
            fwd_r(0, k, True)
            fwd_l(0, k, True)

        xf = x_ref[...]
        scores = jnp.dot(xf, rw_ref[...], preferred_element_type=jnp.float32)
        s_max = jnp.max(scores, axis=-1, keepdims=True)
        p = jnp.exp(scores - s_max)
        probs = p / jnp.sum(p, axis=-1, keepdims=True)
        ids = lax.broadcasted_iota(jnp.int32, (n_tok, n_exp), 1)
        oh0 = ids == idx_ref[:, 0:1]
        oh1 = ids == idx_ref[:, 1:2]
        g0 = jnp.sum(jnp.where(oh0, probs, 0.0), axis=-1, keepdims=True)
        g1 = jnp.sum(jnp.where(oh1, probs, 0.0), axis=-1, keepdims=True)
        gate = probs * (oh0 | oh1).astype(jnp.float32) / (g0 + g1)

        for o in range(N_DEV):
            gate_s[o] = gate[:, N_EXP_LOCAL * o : N_EXP_LOCAL * (o + 1)]

        xb = xf.astype(jnp.bfloat16)
        out_ref[...] = jnp.zeros((n_tok, d_ff), jnp.float32)

        def consume(o, j, w_slot_ref):
            g1c = gate_s[o][:, j : j + 1].astype(jnp.bfloat16)
            out_ref[...] = out_ref[...] + jnp.dot(
                xb * g1c, w_slot_ref, preferred_element_type=jnp.float32
            )

        for k in range(N_SUB):
            consume(me, k, w_r[N_SUB * me + k])
            consume(me, N_SUB + k, w_lv[N_SUB * me + k])

        pending = []
        for h in range(1, N_DEV):
            o_r = lax.rem(me - h + N_DEV, N_DEV)
            o_l = lax.rem(me + h, N_DEV)
            for k in range(N_SUB):
                fwd_r(h - 1, k, False).wait_recv()
                if h <= N_DEV - 2:
                    fwd_r(h, k, True)
                consume(o_r, k, w_r[N_SUB * o_r + k])
            for k in range(N_SUB):
                fwd_l(h - 1, k, False).wait_recv()
                if h <= N_DEV - 2:
                    fwd_l(h, k, True)
                slot = N_SUB * o_l + k
                cp = pltpu.make_async_copy(
                    w_lh.at[slot], w_lv.at[slot],
                    loc_sems.at[N_SUB * (h - 1) + k],
                )
                cp.start()
                pending.append((o_l, k, cp))
            while len(pending) > N_SUB:
                po, pk, pcp = pending.pop(0)
                pcp.wait()
                consume(po, N_SUB + pk, w_lv[N_SUB * po + pk])

        for po, pk, pcp in pending:
            pcp.wait()
            consume(po, N_SUB + pk, w_lv[N_SUB * po + pk])

        for g in range(N_DEV - 1):
            for k in range(N_SUB):
                fwd_r(g, k, False).wait_send()
                fwd_l(g, k, False).wait_send()

    out, _ = pl.pallas_call(
        body,
        out_shape=(
            jax.ShapeDtypeStruct((n_tok, d_ff), jnp.float32),
            jax.ShapeDtypeStruct((N_DEV * N_SUB, 256, 512), jnp.bfloat16),
        ),
        in_specs=[
            pl.BlockSpec(memory_space=pltpu.VMEM),
            pl.BlockSpec(memory_space=pltpu.VMEM),
            pl.BlockSpec(memory_space=pltpu.VMEM),
            pl.BlockSpec(memory_space=pltpu.VMEM),
        ],
        out_specs=(
            pl.BlockSpec(memory_space=pltpu.VMEM),
            pl.BlockSpec(memory_space=pltpu.HBM),
        ),
        scratch_shapes=[
            pltpu.VMEM((N_DEV * N_SUB, 256, 512), jnp.bfloat16),
            pltpu.VMEM((N_DEV * N_SUB, 256, 512), jnp.bfloat16),
            pltpu.VMEM((N_DEV, 512, N_EXP_LOCAL), jnp.float32),
            pltpu.SemaphoreType.DMA((N_STEP,)),
            pltpu.SemaphoreType.DMA((N_STEP,)),
            pltpu.SemaphoreType.DMA((N_STEP,)),
            pltpu.SemaphoreType.DMA((N_STEP,)),
            pltpu.SemaphoreType.DMA((N_STEP,)),
        ],
        compiler_params=pltpu.CompilerParams(
            collective_id=0,
            vmem_limit_bytes=100 * 1024 * 1024,
        ),
    )(x, router_W, route_idx, ew_bf)
    return out
